# baseline (speedup 1.0000x reference)
import functools

import jax
import jax.numpy as jnp
from jax import lax
from jax.experimental import pallas as pl
from jax.experimental.pallas import tpu as pltpu
from jax.experimental.pallas import tpu_sc as plsc

N = 10000
E = 320000
H = 256
HH = 128
G = 64
EPS = 1e-5

NC = 2
NS = 16
LANES = 16
N_PAD = 10240
ROWS_PER_TILE = N_PAD // NS
EPW = E // NS
KBLK = 80
RBLK = 2000


def _k1_body(x_ref, wne_ref, bne_ref, wms_ref, wmd_ref, h_ref, t_ref):
    h = jnp.maximum(
        jnp.dot(x_ref[...], wne_ref[...], preferred_element_type=jnp.float32)
        + bne_ref[...], 0.0)
    h_ref[...] = h
    ha = jnp.dot(h, wms_ref[...], preferred_element_type=jnp.float32)
    hb = jnp.dot(h, wmd_ref[...], preferred_element_type=jnp.float32)
    t_ref[0] = ha[:, :HH]
    t_ref[1] = ha[:, HH:]
    t_ref[2] = hb[:, :HH]
    t_ref[3] = hb[:, HH:]


def _node_encode(x, wne, bne, wms, wmd):
    grid = N // RBLK
    full = lambda shape: pl.BlockSpec(shape, lambda i: (0,) * len(shape))
    return pl.pallas_call(
        _k1_body,
        grid=(grid,),
        in_specs=[
            pl.BlockSpec((RBLK, 128), lambda i: (i, 0)),
            full((128, H)), full((1, H)), full((H, H)), full((H, H)),
        ],
        out_specs=[
            pl.BlockSpec((RBLK, H), lambda i: (i, 0)),
            pl.BlockSpec((4, RBLK, HH), lambda i: (0, i, 0)),
        ],
        out_shape=[
            jax.ShapeDtypeStruct((N, H), jnp.float32),
            jax.ShapeDtypeStruct((4, N, HH), jnp.float32),
        ],
    )(x, wne, bne, wms, wmd)


def _k2_body(ea_ref, w0_ref, b0_ref, w1_ref, b1_ref, w2_ref, b2_ref,
             w3_ref, b3_ref, wme_ref, bm_ref, c_ref):
    f32 = jnp.float32
    bf = jnp.bfloat16
    a = jnp.maximum(jnp.dot(ea_ref[...], w0_ref[...], preferred_element_type=f32)
                    + b0_ref[...], 0.0)
    a = jnp.maximum(jnp.dot(a.astype(bf), w1_ref[...],
                            preferred_element_type=f32) + b1_ref[...], 0.0)
    a = jnp.maximum(jnp.dot(a.astype(bf), w2_ref[...],
                            preferred_element_type=f32) + b2_ref[...], 0.0)
    e = (jnp.dot(a.astype(bf), w3_ref[...], preferred_element_type=f32)
         + b3_ref[...])
    eb = e.astype(bf)
    for l in range(3):
        cl = (jnp.dot(eb, wme_ref[l], preferred_element_type=f32)
              + bm_ref[pl.ds(l, 1)])
        c_ref[l, 0] = cl[:, :HH]
        c_ref[l, 1] = cl[:, HH:]


def _edge_encode(ea, w0, b0, w1, b1, w2, b2, w3, b3, wme, bm):
    grid = E // RBLK
    full = lambda shape: pl.BlockSpec(shape, lambda i: (0,) * len(shape))
    return pl.pallas_call(
        _k2_body,
        grid=(grid,),
        in_specs=[
            pl.BlockSpec((RBLK, 16), lambda i: (i, 0)),
            full((16, 512)), full((1, 512)),
            full((512, 512)), full((1, 512)),
            full((512, 512)), full((1, 512)),
            full((512, H)), full((1, H)),
            full((3, H, H)), full((3, H)),
        ],
        out_specs=pl.BlockSpec((3, 2, RBLK, HH), lambda i: (0, 0, i, 0)),
        out_shape=jax.ShapeDtypeStruct((3, 2, E, HH), jnp.float32),
    )(ea, w0, b0, w1, b1, w2, b2, w3, b3, wme, bm)


def _ku_body(h_ref, a0_ref, a1_ref, wuh_ref, wua_ref, wub_ref, bu_ref,
             wms_ref, wmd_ref, hn_ref, t_ref):
    f32 = jnp.float32
    hn = jnp.dot(h_ref[...], wuh_ref[...], preferred_element_type=f32)
    hn += jnp.dot(a0_ref[...], wua_ref[...], preferred_element_type=f32)
    hn += jnp.dot(a1_ref[...], wub_ref[...], preferred_element_type=f32)
    hn = jnp.maximum(hn + bu_ref[...], 0.0)
    hn_ref[...] = hn
    if t_ref is not None:
        ha = jnp.dot(hn, wms_ref[...], preferred_element_type=f32)
        hb = jnp.dot(hn, wmd_ref[...], preferred_element_type=f32)
        t_ref[0] = ha[:, :HH]
        t_ref[1] = ha[:, HH:]
        t_ref[2] = hb[:, :HH]
        t_ref[3] = hb[:, HH:]


def _update(h, a0, a1, wuh, wua, wub, bu, wms=None, wmd=None):
    grid = N // RBLK
    full = lambda shape: pl.BlockSpec(shape, lambda i: (0,) * len(shape))
    with_tab = wms is not None
    in_specs = [
        pl.BlockSpec((RBLK, H), lambda i: (i, 0)),
        pl.BlockSpec((RBLK, HH), lambda i: (i, 0)),
        pl.BlockSpec((RBLK, HH), lambda i: (i, 0)),
        full((H, H)), full((HH, H)), full((HH, H)), full((1, H)),
    ]
    out_specs = [pl.BlockSpec((RBLK, H), lambda i: (i, 0))]
    out_shape = [jax.ShapeDtypeStruct((N, H), jnp.float32)]
    args = [h, a0, a1, wuh, wua, wub, bu]
    if with_tab:
        in_specs += [full((H, H)), full((H, H))]
        out_specs.append(pl.BlockSpec((4, RBLK, HH), lambda i: (0, i, 0)))
        out_shape.append(jax.ShapeDtypeStruct((4, N, HH), jnp.float32))
        args += [wms, wmd]
        body = _ku_body
    else:
        body = lambda *rs: _ku_body(*rs[:7], None, None, rs[7], None)
    return pl.pallas_call(
        body,
        grid=(grid,),
        in_specs=in_specs,
        out_specs=out_specs,
        out_shape=out_shape,
    )(*args)


def _k3_body(h_ref, b_ref, wd0_ref, bd0_ref, wd1_ref, bd1_ref, out_ref,
             sum_ref, cnt_ref):
    i = pl.program_id(0)
    nblk = pl.num_programs(0)

    @pl.when(i == 0)
    def _init():
        sum_ref[...] = jnp.zeros_like(sum_ref)
        cnt_ref[...] = jnp.zeros_like(cnt_ref)

    bb = b_ref[0]
    gid = lax.broadcasted_iota(jnp.int32, (G, RBLK), 0)
    p = jnp.where(gid == bb, 1.0, 0.0)
    sum_ref[...] += jnp.dot(p, h_ref[...], preferred_element_type=jnp.float32)
    cnt_ref[...] += jnp.sum(p, axis=1, keepdims=True)

    @pl.when(i == nblk - 1)
    def _fin():
        pooled = sum_ref[...] / jnp.maximum(cnt_ref[...], 1.0)
        z = jnp.maximum(
            jnp.dot(pooled, wd0_ref[...], preferred_element_type=jnp.float32)
            + bd0_ref[...], 0.0)
        out_ref[...] = (jnp.dot(z, wd1_ref[...],
                                preferred_element_type=jnp.float32)
                        + bd1_ref[...])


def _pool_decode(h, batchf, wd0, bd0, wd1, bd1):
    grid = N // RBLK
    full = lambda shape: pl.BlockSpec(shape, lambda i: (0,) * len(shape))
    return pl.pallas_call(
        _k3_body,
        grid=(grid,),
        in_specs=[
            pl.BlockSpec((RBLK, H), lambda i: (i, 0)),
            pl.BlockSpec((1, 1, RBLK), lambda i: (i, 0, 0)),
            full((H, 512)), full((1, 512)), full((512, 1)), full((1, 1)),
        ],
        out_specs=pl.BlockSpec((G, 1), lambda i: (0, 0)),
        out_shape=jax.ShapeDtypeStruct((G, 1), jnp.float32),
        scratch_shapes=[
            pltpu.VMEM((G, H), jnp.float32),
            pltpu.VMEM((G, 1), jnp.float32),
        ],
    )(h, batchf, wd0, bd0, wd1, bd1)


def _sc_edge_body(idx_h, tab_h, c_h, out_h,
                  idxb, srcv, dgv, buf_a, buf_b, buf_c, spmem,
                  sem_i, sem_a, sem_b, sem_c):
    cid = lax.axis_index("c")
    sid = lax.axis_index("s")

    def _zrow(i, carry):
        for j in range(HH // LANES):
            buf_a[i, pl.ds(j * LANES, LANES)] = jnp.zeros((LANES,),
                                                          jnp.float32)
        return carry
    lax.fori_loop(0, KBLK, _zrow, 0)
    for q in range(ROWS_PER_TILE // KBLK):
        pltpu.sync_copy(
            buf_a, spmem.at[pl.ds(sid * ROWS_PER_TILE + q * KBLK, KBLK)])
    plsc.subcore_barrier()

    src_off = cid * N
    dst_off = (2 + cid) * N
    c_base = cid * E
    nblk = EPW // KBLK

    def _block(b, carry):
        gb = sid * nblk + b
        pltpu.sync_copy(idx_h.at[gb], idxb)
        for j in range(KBLK // LANES):
            sl = pl.ds(j * LANES, LANES)
            srcv[sl] = idxb[0, sl] + src_off
            dgv[sl] = idxb[1, sl] + dst_off
        cp_a = pltpu.async_copy(tab_h.at[srcv], buf_a, sem_a)
        cp_b = pltpu.async_copy(tab_h.at[dgv], buf_b, sem_b)
        cp_c = pltpu.async_copy(
            c_h.at[pl.ds(c_base + sid * EPW + b * KBLK, KBLK)], buf_c, sem_c)
        cp_a.wait()
        cp_b.wait()
        cp_c.wait()

        def _row(i, rc):
            for j in range(HH // LANES):
                sl = pl.ds(j * LANES, LANES)
                v = buf_a[i, sl] + buf_b[i, sl] + buf_c[i, sl]
                buf_a[i, sl] = jnp.maximum(v, 0.0)
            return rc
        lax.fori_loop(0, KBLK, _row, 0)

        pltpu.sync_copy(buf_a, spmem.at[idxb.at[1]], add=True)
        return carry

    lax.fori_loop(0, nblk, _block, 0)

    plsc.subcore_barrier()
    pltpu.sync_copy(
        spmem.at[pl.ds(sid * ROWS_PER_TILE, ROWS_PER_TILE)],
        out_h.at[pl.ds(cid * N_PAD + sid * ROWS_PER_TILE, ROWS_PER_TILE)])


def _sc_edge_stage(idx_blk, tab2, c2):
    mesh = plsc.VectorSubcoreMesh(core_axis_name="c", subcore_axis_name="s")
    return pl.kernel(
        _sc_edge_body,
        out_type=jax.ShapeDtypeStruct((NC * N_PAD, HH), jnp.float32),
        mesh=mesh,
        scratch_types=[
            pltpu.VMEM((2, KBLK), jnp.int32),
            pltpu.VMEM((KBLK,), jnp.int32),
            pltpu.VMEM((KBLK,), jnp.int32),
            pltpu.VMEM((KBLK, HH), jnp.float32),
            pltpu.VMEM((KBLK, HH), jnp.float32),
            pltpu.VMEM((KBLK, HH), jnp.float32),
            pltpu.VMEM_SHARED((N_PAD, HH), jnp.float32),
            pltpu.SemaphoreType.DMA,
            pltpu.SemaphoreType.DMA,
            pltpu.SemaphoreType.DMA,
            pltpu.SemaphoreType.DMA,
        ],
    )(idx_blk, tab2, c2)


def kernel(x, edge_index, edge_attr, batch, params):
    p = params
    f32 = jnp.float32

    inv = 1.0 / jnp.sqrt(jnp.float32(1.0 + EPS))
    s0, t0 = p['bng0'] * inv, p['bnb0']
    s1, t1 = p['bng1'] * inv, p['bnb1']
    s2, t2 = p['bng2'] * inv, p['bnb2']
    w0, b0 = p['We0'], p['be0']
    w1 = s0[:, None] * p['We1']
    b1 = p['be1'] + t0 @ p['We1']
    w2 = s1[:, None] * p['We2']
    b2 = p['be2'] + t1 @ p['We2']
    w3 = s2[:, None] * p['We3']
    b3 = p['be3'] + t2 @ p['We3']

    wms = [p[f'Wm{l}'][:H] for l in range(3)]
    wmd = [p[f'Wm{l}'][H:2 * H] for l in range(3)]
    wme = jnp.stack([p[f'Wm{l}'][2 * H:] for l in range(3)])
    bm = jnp.stack([p[f'bm{l}'] for l in range(3)])
    wuh = [p[f'Wu{l}'][:H] for l in range(3)]
    wua = [p[f'Wu{l}'][H:H + HH] for l in range(3)]
    wub = [p[f'Wu{l}'][H + HH:] for l in range(3)]

    idx_blk = jnp.transpose(
        edge_index.astype(jnp.int32).reshape(2, E // KBLK, KBLK), (1, 0, 2))

    h, tab = _node_encode(x, p['W_ne'], p['b_ne'].reshape(1, H).astype(f32),
                          wms[0], wmd[0])

    bf = jnp.bfloat16
    c_all = _edge_encode(
        edge_attr, w0, b0.reshape(1, 512), w1.astype(bf), b1.reshape(1, 512),
        w2.astype(bf), b2.reshape(1, 512), w3.astype(bf), b3.reshape(1, H),
        wme.astype(bf), bm)

    for l in range(3):
        tab2 = tab.reshape(4 * N, HH)
        c2 = c_all[l].reshape(2 * E, HH)
        aggr2 = _sc_edge_stage(idx_blk, tab2, c2)
        a0 = aggr2[:N]
        a1 = aggr2[N_PAD:N_PAD + N]
        if l < 2:
            h, tab = _update(h, a0, a1, wuh[l], wua[l], wub[l],
                             p[f'bu{l}'].reshape(1, H),
                             wms[l + 1], wmd[l + 1])
        else:
            (h,) = _update(h, a0, a1, wuh[l], wua[l], wub[l],
                           p[f'bu{l}'].reshape(1, H))

    batchf = batch.astype(jnp.int32).reshape(N // RBLK, 1, RBLK)
    out = _pool_decode(h, batchf, p['Wd0'], p['bd0'].reshape(1, 512),
                       p['Wd1'], p['bd1'].reshape(1, 1))
    return out


# speedup vs baseline: 1.1540x; 1.1540x over previous
import functools

import jax
import jax.numpy as jnp
from jax import lax
from jax.experimental import pallas as pl
from jax.experimental.pallas import tpu as pltpu
from jax.experimental.pallas import tpu_sc as plsc

N = 10000
E = 320000
H = 256
HH = 128
G = 64
EPS = 1e-5

NC = 2
NS = 16
LANES = 16
N_PAD = 10240
ROWS_PER_TILE = N_PAD // NS
EPW = E // NS
KBLK = 80
RBLK = 2000


def _k1_body(x_ref, wne_ref, bne_ref, wms_ref, wmd_ref, h_ref, t_ref):
    h = jnp.maximum(
        jnp.dot(x_ref[...], wne_ref[...], preferred_element_type=jnp.float32)
        + bne_ref[...], 0.0)
    h_ref[...] = h
    ha = jnp.dot(h, wms_ref[...], preferred_element_type=jnp.float32)
    hb = jnp.dot(h, wmd_ref[...], preferred_element_type=jnp.float32)
    t_ref[0] = ha[:, :HH]
    t_ref[1] = ha[:, HH:]
    t_ref[2] = hb[:, :HH]
    t_ref[3] = hb[:, HH:]


def _node_encode(x, wne, bne, wms, wmd):
    grid = N // RBLK
    full = lambda shape: pl.BlockSpec(shape, lambda i: (0,) * len(shape))
    return pl.pallas_call(
        _k1_body,
        grid=(grid,),
        in_specs=[
            pl.BlockSpec((RBLK, 128), lambda i: (i, 0)),
            full((128, H)), full((1, H)), full((H, H)), full((H, H)),
        ],
        out_specs=[
            pl.BlockSpec((RBLK, H), lambda i: (i, 0)),
            pl.BlockSpec((4, RBLK, HH), lambda i: (0, i, 0)),
        ],
        out_shape=[
            jax.ShapeDtypeStruct((N, H), jnp.float32),
            jax.ShapeDtypeStruct((4, N, HH), jnp.float32),
        ],
    )(x, wne, bne, wms, wmd)


def _k2_body(ea_ref, w0_ref, b0_ref, w1_ref, b1_ref, w2_ref, b2_ref,
             w3_ref, b3_ref, wme_ref, bm_ref, c_ref):
    f32 = jnp.float32
    bf = jnp.bfloat16
    a = jnp.maximum(jnp.dot(ea_ref[...], w0_ref[...], preferred_element_type=f32)
                    + b0_ref[...], 0.0)
    a = jnp.maximum(jnp.dot(a.astype(bf), w1_ref[...],
                            preferred_element_type=f32) + b1_ref[...], 0.0)
    a = jnp.maximum(jnp.dot(a.astype(bf), w2_ref[...],
                            preferred_element_type=f32) + b2_ref[...], 0.0)
    e = (jnp.dot(a.astype(bf), w3_ref[...], preferred_element_type=f32)
         + b3_ref[...])
    eb = e.astype(bf)
    for l in range(3):
        cl = (jnp.dot(eb, wme_ref[l], preferred_element_type=f32)
              + bm_ref[pl.ds(l, 1)])
        c_ref[l, 0] = cl[:, :HH]
        c_ref[l, 1] = cl[:, HH:]


def _edge_encode(ea, w0, b0, w1, b1, w2, b2, w3, b3, wme, bm):
    grid = E // RBLK
    full = lambda shape: pl.BlockSpec(shape, lambda i: (0,) * len(shape))
    return pl.pallas_call(
        _k2_body,
        grid=(grid,),
        in_specs=[
            pl.BlockSpec((RBLK, 16), lambda i: (i, 0)),
            full((16, 512)), full((1, 512)),
            full((512, 512)), full((1, 512)),
            full((512, 512)), full((1, 512)),
            full((512, H)), full((1, H)),
            full((3, H, H)), full((3, H)),
        ],
        out_specs=pl.BlockSpec((3, 2, RBLK, HH), lambda i: (0, 0, i, 0)),
        out_shape=jax.ShapeDtypeStruct((3, 2, E, HH), jnp.float32),
    )(ea, w0, b0, w1, b1, w2, b2, w3, b3, wme, bm)


def _ku_body(h_ref, a0_ref, a1_ref, wuh_ref, wua_ref, wub_ref, bu_ref,
             wms_ref, wmd_ref, hn_ref, t_ref):
    f32 = jnp.float32
    hn = jnp.dot(h_ref[...], wuh_ref[...], preferred_element_type=f32)
    hn += jnp.dot(a0_ref[...], wua_ref[...], preferred_element_type=f32)
    hn += jnp.dot(a1_ref[...], wub_ref[...], preferred_element_type=f32)
    hn = jnp.maximum(hn + bu_ref[...], 0.0)
    hn_ref[...] = hn
    if t_ref is not None:
        ha = jnp.dot(hn, wms_ref[...], preferred_element_type=f32)
        hb = jnp.dot(hn, wmd_ref[...], preferred_element_type=f32)
        t_ref[0] = ha[:, :HH]
        t_ref[1] = ha[:, HH:]
        t_ref[2] = hb[:, :HH]
        t_ref[3] = hb[:, HH:]


def _update(h, a0, a1, wuh, wua, wub, bu, wms=None, wmd=None):
    grid = N // RBLK
    full = lambda shape: pl.BlockSpec(shape, lambda i: (0,) * len(shape))
    with_tab = wms is not None
    in_specs = [
        pl.BlockSpec((RBLK, H), lambda i: (i, 0)),
        pl.BlockSpec((RBLK, HH), lambda i: (i, 0)),
        pl.BlockSpec((RBLK, HH), lambda i: (i, 0)),
        full((H, H)), full((HH, H)), full((HH, H)), full((1, H)),
    ]
    out_specs = [pl.BlockSpec((RBLK, H), lambda i: (i, 0))]
    out_shape = [jax.ShapeDtypeStruct((N, H), jnp.float32)]
    args = [h, a0, a1, wuh, wua, wub, bu]
    if with_tab:
        in_specs += [full((H, H)), full((H, H))]
        out_specs.append(pl.BlockSpec((4, RBLK, HH), lambda i: (0, i, 0)))
        out_shape.append(jax.ShapeDtypeStruct((4, N, HH), jnp.float32))
        args += [wms, wmd]
        body = _ku_body
    else:
        body = lambda *rs: _ku_body(*rs[:7], None, None, rs[7], None)
    return pl.pallas_call(
        body,
        grid=(grid,),
        in_specs=in_specs,
        out_specs=out_specs,
        out_shape=out_shape,
    )(*args)


def _k3_body(h_ref, b_ref, wd0_ref, bd0_ref, wd1_ref, bd1_ref, out_ref,
             sum_ref, cnt_ref):
    i = pl.program_id(0)
    nblk = pl.num_programs(0)

    @pl.when(i == 0)
    def _init():
        sum_ref[...] = jnp.zeros_like(sum_ref)
        cnt_ref[...] = jnp.zeros_like(cnt_ref)

    bb = b_ref[0]
    gid = lax.broadcasted_iota(jnp.int32, (G, RBLK), 0)
    p = jnp.where(gid == bb, 1.0, 0.0)
    sum_ref[...] += jnp.dot(p, h_ref[...], preferred_element_type=jnp.float32)
    cnt_ref[...] += jnp.sum(p, axis=1, keepdims=True)

    @pl.when(i == nblk - 1)
    def _fin():
        pooled = sum_ref[...] / jnp.maximum(cnt_ref[...], 1.0)
        z = jnp.maximum(
            jnp.dot(pooled, wd0_ref[...], preferred_element_type=jnp.float32)
            + bd0_ref[...], 0.0)
        out_ref[...] = (jnp.dot(z, wd1_ref[...],
                                preferred_element_type=jnp.float32)
                        + bd1_ref[...])


def _pool_decode(h, batchf, wd0, bd0, wd1, bd1):
    grid = N // RBLK
    full = lambda shape: pl.BlockSpec(shape, lambda i: (0,) * len(shape))
    return pl.pallas_call(
        _k3_body,
        grid=(grid,),
        in_specs=[
            pl.BlockSpec((RBLK, H), lambda i: (i, 0)),
            pl.BlockSpec((1, 1, RBLK), lambda i: (i, 0, 0)),
            full((H, 512)), full((1, 512)), full((512, 1)), full((1, 1)),
        ],
        out_specs=pl.BlockSpec((G, 1), lambda i: (0, 0)),
        out_shape=jax.ShapeDtypeStruct((G, 1), jnp.float32),
        scratch_shapes=[
            pltpu.VMEM((G, H), jnp.float32),
            pltpu.VMEM((G, 1), jnp.float32),
        ],
    )(h, batchf, wd0, bd0, wd1, bd1)


def _sc_edge_body(layer, idx_h, tab_h, c_h, out_h,
                  idxb, srcv, dgv, buf_a, buf_b, buf_c, spmem,
                  sem_i, sem_a, sem_b, sem_c):
    cid = lax.axis_index("c")
    sid = lax.axis_index("s")

    def _zrow(i, carry):
        for j in range(HH // LANES):
            buf_a[i, pl.ds(j * LANES, LANES)] = jnp.zeros((LANES,),
                                                          jnp.float32)
        return carry
    lax.fori_loop(0, KBLK, _zrow, 0)
    for q in range(ROWS_PER_TILE // KBLK):
        pltpu.sync_copy(
            buf_a, spmem.at[pl.ds(sid * ROWS_PER_TILE + q * KBLK, KBLK)])
    plsc.subcore_barrier()

    src_off = cid * N
    dst_off = (2 + cid) * N
    nblk = EPW // KBLK

    def _block(b, carry):
        base = sid * EPW + b * KBLK
        pltpu.sync_copy(idx_h.at[sid * nblk + b], idxb)
        for j in range(KBLK // LANES):
            sl = pl.ds(j * LANES, LANES)
            srcv[sl] = idxb[0, sl] + src_off
            dgv[sl] = idxb[1, sl] + dst_off
        cp_a = pltpu.async_copy(tab_h.at[srcv], buf_a, sem_a)
        cp_b = pltpu.async_copy(tab_h.at[dgv], buf_b, sem_b)
        cp_c = pltpu.async_copy(
            c_h.at[layer, cid, pl.ds(base, KBLK)], buf_c, sem_c)
        cp_a.wait()
        cp_b.wait()
        cp_c.wait()

        def _row(i, rc):
            for j in range(HH // LANES):
                sl = pl.ds(j * LANES, LANES)
                v = buf_a[i, sl] + buf_b[i, sl] + buf_c[i, sl]
                buf_a[i, sl] = jnp.maximum(v, 0.0)
            return rc
        lax.fori_loop(0, KBLK, _row, 0)

        pltpu.sync_copy(buf_a, spmem.at[idxb.at[1]], add=True)
        return carry

    lax.fori_loop(0, nblk, _block, 0)

    plsc.subcore_barrier()
    pltpu.sync_copy(
        spmem.at[pl.ds(sid * ROWS_PER_TILE, ROWS_PER_TILE)],
        out_h.at[pl.ds(cid * N_PAD + sid * ROWS_PER_TILE, ROWS_PER_TILE)])


def _sc_edge_stage(idx2, tab2, c_all, layer):
    mesh = plsc.VectorSubcoreMesh(core_axis_name="c", subcore_axis_name="s")
    return pl.kernel(
        functools.partial(_sc_edge_body, layer),
        out_type=jax.ShapeDtypeStruct((NC * N_PAD, HH), jnp.float32),
        mesh=mesh,
        scratch_types=[
            pltpu.VMEM((2, KBLK), jnp.int32),
            pltpu.VMEM((KBLK,), jnp.int32),
            pltpu.VMEM((KBLK,), jnp.int32),
            pltpu.VMEM((KBLK, HH), jnp.float32),
            pltpu.VMEM((KBLK, HH), jnp.float32),
            pltpu.VMEM((KBLK, HH), jnp.float32),
            pltpu.VMEM_SHARED((N_PAD, HH), jnp.float32),
            pltpu.SemaphoreType.DMA,
            pltpu.SemaphoreType.DMA,
            pltpu.SemaphoreType.DMA,
            pltpu.SemaphoreType.DMA,
        ],
    )(idx2, tab2, c_all)


def kernel(x, edge_index, edge_attr, batch, params):
    p = params
    f32 = jnp.float32

    inv = 1.0 / jnp.sqrt(jnp.float32(1.0 + EPS))
    s0, t0 = p['bng0'] * inv, p['bnb0']
    s1, t1 = p['bng1'] * inv, p['bnb1']
    s2, t2 = p['bng2'] * inv, p['bnb2']
    w0, b0 = p['We0'], p['be0']
    w1 = s0[:, None] * p['We1']
    b1 = p['be1'] + t0 @ p['We1']
    w2 = s1[:, None] * p['We2']
    b2 = p['be2'] + t1 @ p['We2']
    w3 = s2[:, None] * p['We3']
    b3 = p['be3'] + t2 @ p['We3']

    wms = [p[f'Wm{l}'][:H] for l in range(3)]
    wmd = [p[f'Wm{l}'][H:2 * H] for l in range(3)]
    wme = jnp.stack([p[f'Wm{l}'][2 * H:] for l in range(3)])
    bm = jnp.stack([p[f'bm{l}'] for l in range(3)])
    wuh = [p[f'Wu{l}'][:H] for l in range(3)]
    wua = [p[f'Wu{l}'][H:H + HH] for l in range(3)]
    wub = [p[f'Wu{l}'][H + HH:] for l in range(3)]

    idx2 = jnp.transpose(
        edge_index.astype(jnp.int32).reshape(2, E // KBLK, KBLK), (1, 0, 2))

    h, tab = _node_encode(x, p['W_ne'], p['b_ne'].reshape(1, H).astype(f32),
                          wms[0], wmd[0])

    bf = jnp.bfloat16
    c_all = _edge_encode(
        edge_attr, w0, b0.reshape(1, 512), w1.astype(bf), b1.reshape(1, 512),
        w2.astype(bf), b2.reshape(1, 512), w3.astype(bf), b3.reshape(1, H),
        wme.astype(bf), bm)

    for l in range(3):
        tab2 = tab.reshape(4 * N, HH)
        aggr2 = _sc_edge_stage(idx2, tab2, c_all, l)
        a0 = aggr2[:N]
        a1 = aggr2[N_PAD:N_PAD + N]
        if l < 2:
            h, tab = _update(h, a0, a1, wuh[l], wua[l], wub[l],
                             p[f'bu{l}'].reshape(1, H),
                             wms[l + 1], wmd[l + 1])
        else:
            (h,) = _update(h, a0, a1, wuh[l], wua[l], wub[l],
                           p[f'bu{l}'].reshape(1, H))

    batchf = batch.astype(jnp.int32).reshape(N // RBLK, 1, RBLK)
    out = _pool_decode(h, batchf, p['Wd0'], p['bd0'].reshape(1, 512),
                       p['Wd1'], p['bd1'].reshape(1, 1))
    return out
